# baseline (speedup 1.0000x reference)
import functools

import jax
import jax.numpy as jnp
from jax import lax
from jax.experimental import pallas as pl
from jax.experimental.pallas import tpu as pltpu
from jax.experimental.pallas import tpu_sc as plsc

NUM_INPUTS = 32
NUM_GROUPS = 1000000
BATCH = 16384
NC = 2
NS = 16
NW = NC * NS
BPW = BATCH // NW
STEP = 8
NSTEP = BPW // STEP
NSLOT = 3
LANE = 128
HALF = NUM_INPUTS // 2


def _sc_body(xt_ref, ids_ref, wb_ref, tab_ref, out_ref,
             ids_v, xt_v, cols_v, out_v, wb_v, dupi_v, dupf_v, sem0, sem1):
    wid = lax.axis_index("s") * NC + lax.axis_index("c")
    base = wid * BPW

    pltpu.sync_copy(ids_ref.at[pl.ds(base, BPW)],
                    ids_v.at[pl.ds(0, BPW)])
    pltpu.sync_copy(xt_ref.at[:, pl.ds(base, BPW)], xt_v)
    pltpu.sync_copy(wb_ref, wb_v)

    lanes = lax.iota(jnp.int32, 16)
    low = lanes < STEP
    k8 = lax.rem(lanes, STEP)

    def issue(s, slot, sem):
        idv = ids_v[pl.ds(s * STEP, 16)]
        colv = idv // LANE
        for k in range(STEP):
            off = pl.multiple_of(colv[k] * LANE, LANE)
            pltpu.async_copy(
                tab_ref.at[:, pl.ds(off, LANE)],
                cols_v.at[slot, k],
                sem)

    def drain(sem):
        for _k in range(STEP):
            pltpu.make_async_copy(
                tab_ref.at[:, pl.ds(0, LANE)], cols_v.at[0, 0], sem).wait()

    issue(0, 0, sem0)

    def step(s, _):
        odd = lax.rem(s, 2)
        slot = lax.rem(s, NSLOT)

        @pl.when(s < NSTEP - 1)
        def _():
            nslot = lax.rem(s + 1, NSLOT)

            @pl.when(odd == 0)
            def _():
                issue(s + 1, nslot, sem1)

            @pl.when(odd == 1)
            def _():
                issue(s + 1, nslot, sem0)

        @pl.when(odd == 0)
        def _():
            drain(sem0)

        @pl.when(odd == 1)
        def _():
            drain(sem1)

        o = s * STEP
        idv = ids_v[pl.ds(o, 16)]
        dupi_v[pl.ds(0, 16)] = idv
        dupi_v[pl.ds(STEP, 16)] = idv
        iddup = dupi_v[pl.ds(0, 16)]
        loff = lax.rem(iddup, LANE)
        posdup = o + k8
        slotv = jnp.broadcast_to(slot, (16,))

        acc = jnp.zeros((16,), jnp.float32)
        for j in range(HALF):
            jvec = jnp.where(low, j, j + HALF)
            wv = plsc.load_gather(cols_v, [slotv, k8, jvec, loff])
            xv = plsc.load_gather(xt_v, [jvec, posdup])
            acc = acc + xv * (wv + wb_v[j, 0:16])

        dupf_v[pl.ds(0, 16)] = acc
        t1 = dupf_v[pl.ds(0, 16)]
        t2 = dupf_v[pl.ds(STEP, 16)]
        tot = t1 + t2
        plsc.store_scatter(out_v, [o + k8], tot, mask=low)
        return 0

    lax.fori_loop(0, NSTEP, step, 0)

    pltpu.sync_copy(out_v, out_ref.at[pl.ds(base, BPW)])


@jax.jit
def _run(XT, ids, wb, tabT):
    mesh = plsc.VectorSubcoreMesh(core_axis_name="c", subcore_axis_name="s")
    f = functools.partial(
        pl.kernel,
        out_type=jax.ShapeDtypeStruct((BATCH,), jnp.float32),
        mesh=mesh,
        compiler_params=pltpu.CompilerParams(needs_layout_passes=False),
        scratch_types=[
            pltpu.VMEM((BPW + 16,), jnp.int32),
            pltpu.VMEM((NUM_INPUTS, BPW), jnp.float32),
            pltpu.VMEM((NSLOT, STEP, NUM_INPUTS, LANE), jnp.float32),
            pltpu.VMEM((BPW,), jnp.float32),
            pltpu.VMEM((HALF, 16), jnp.float32),
            pltpu.VMEM((STEP + 16,), jnp.int32),
            pltpu.VMEM((STEP + 16,), jnp.float32),
            pltpu.SemaphoreType.DMA,
            pltpu.SemaphoreType.DMA,
        ],
    )(_sc_body)
    return f(XT, ids, wb, tabT)


def kernel(X, ids, W_weight, W_random):
    ids = ids.astype(jnp.int32)
    XT = jnp.transpose(X)
    tabT = jnp.transpose(W_random)
    ww = W_weight.reshape(NUM_INPUTS)
    wb = jnp.where(jnp.arange(16)[None, :] < STEP,
                   ww[:HALF, None], ww[HALF:, None])
    return _run(XT, ids, wb, tabT)


# speedup vs baseline: 1.0139x; 1.0139x over previous
import functools

import jax
import jax.numpy as jnp
from jax import lax
from jax.experimental import pallas as pl
from jax.experimental.pallas import tpu as pltpu
from jax.experimental.pallas import tpu_sc as plsc

NUM_INPUTS = 32
NUM_GROUPS = 1000000
BATCH = 16384
NC = 2
NS = 16
NW = NC * NS
BPW = BATCH // NW
GRP = 16
NGRP = BPW // GRP
LANE = 128


def _sc_body(xt_ref, ids_ref, wb_ref, tab_ref, out_ref,
             ids_v, xt_v, cols_v, out_v, wb_v, sem):
    wid = lax.axis_index("s") * NC + lax.axis_index("c")
    base = wid * BPW

    pltpu.sync_copy(ids_ref.at[pl.ds(base, BPW)], ids_v)
    pltpu.sync_copy(xt_ref.at[:, pl.ds(base, BPW)], xt_v)
    pltpu.sync_copy(wb_ref, wb_v)

    lanes = lax.iota(jnp.int32, GRP)

    def group(g, _):
        o = g * GRP
        idv = ids_v[pl.ds(o, GRP)]
        colv = idv // LANE
        loff = idv - colv * LANE

        copies = []
        for k in range(GRP):
            off = pl.multiple_of(colv[k] * LANE, LANE)
            copies.append(pltpu.async_copy(
                tab_ref.at[:, pl.ds(off, LANE)],
                cols_v.at[k],
                sem))
        for cp in copies:
            cp.wait()

        acc = jnp.zeros((GRP,), jnp.float32)
        for j in range(NUM_INPUTS):
            wv = plsc.load_gather(
                cols_v, [lanes, jnp.full((GRP,), j, jnp.int32), loff])
            xv = xt_v[j, pl.ds(o, GRP)]
            acc = acc + xv * (wv + wb_v[j, 0:GRP])
        out_v[pl.ds(o, GRP)] = acc
        return 0

    lax.fori_loop(0, NGRP, group, 0)

    pltpu.sync_copy(out_v, out_ref.at[pl.ds(base, BPW)])


@jax.jit
def _run(XT, ids, wb, tabT):
    mesh = plsc.VectorSubcoreMesh(core_axis_name="c", subcore_axis_name="s")
    f = functools.partial(
        pl.kernel,
        out_type=jax.ShapeDtypeStruct((BATCH,), jnp.float32),
        mesh=mesh,
        compiler_params=pltpu.CompilerParams(needs_layout_passes=False),
        scratch_types=[
            pltpu.VMEM((BPW,), jnp.int32),
            pltpu.VMEM((NUM_INPUTS, BPW), jnp.float32),
            pltpu.VMEM((GRP, NUM_INPUTS, LANE), jnp.float32),
            pltpu.VMEM((BPW,), jnp.float32),
            pltpu.VMEM((NUM_INPUTS, GRP), jnp.float32),
            pltpu.SemaphoreType.DMA,
        ],
    )(_sc_body)
    return f(XT, ids, wb, tabT)


def kernel(X, ids, W_weight, W_random):
    ids = ids.astype(jnp.int32)
    XT = jnp.transpose(X)
    tabT = jnp.transpose(W_random)
    wb = jnp.broadcast_to(W_weight.reshape(NUM_INPUTS, 1), (NUM_INPUTS, GRP))
    return _run(XT, ids, wb, tabT)
